# baseline (speedup 1.0000x reference)
import jax
import jax.numpy as jnp
from jax.experimental import pallas as pl
from jax.experimental.pallas import tpu as pltpu


_NQ = 4


def _packed_ffn_kernel(hbm_x_ref, w1_ref, w2_ref, o_ref, xb_ref, sem_ref):
    i = pl.program_id(0)
    n = pl.num_programs(0)
    R = o_ref.shape[0]
    Q = R // _NQ
    slot = jax.lax.rem(i, 2)
    nxt = jax.lax.rem(i + 1, 2)

    def _stripe(step, s, q):
        return pltpu.make_async_copy(
            hbm_x_ref.at[pl.ds(step * R + q * Q, Q), :],
            xb_ref.at[s, pl.ds(q * Q, Q), :],
            sem_ref.at[s, q],
        )

    @pl.when(i == 0)
    def _():
        for q in range(_NQ):
            _stripe(0, 0, q).start()

    @pl.when(i + 1 < n)
    def _():
        for q in range(_NQ):
            _stripe(i + 1, nxt, q).start()

    for q in range(_NQ):
        _stripe(i, slot, q).wait()

    w1 = w1_ref[...].astype(jnp.bfloat16)
    w2 = w2_ref[...].astype(jnp.bfloat16)
    z2 = jnp.zeros_like(w2)
    w2p = jnp.concatenate(
        [jnp.concatenate([w2, z2], axis=1), jnp.concatenate([z2, w2], axis=1)],
        axis=0,
    )
    x = xb_ref[slot].astype(jnp.bfloat16)
    h = jnp.tanh(jnp.dot(x, w1, preferred_element_type=jnp.float32))
    hp = jnp.concatenate(
        [h[: R // 2].astype(jnp.bfloat16), h[R // 2 :].astype(jnp.bfloat16)],
        axis=1,
    )
    y = jnp.dot(hp, w2p, preferred_element_type=jnp.float32)
    o_ref[: R // 2, :] = y[:, :128]
    o_ref[R // 2 :, :] = y[:, 128:]


def kernel(label_emb, w1_cat, w2_bd):
    L, E = label_emb.shape
    HE = w1_cat.shape[1]

    R = 16384
    while L % R:
        R //= 2

    return pl.pallas_call(
        _packed_ffn_kernel,
        out_shape=jax.ShapeDtypeStruct((L, HE), label_emb.dtype),
        grid=(L // R,),
        in_specs=[
            pl.BlockSpec(memory_space=pltpu.MemorySpace.HBM),
            pl.BlockSpec((E, HE), lambda i: (0, 0)),
            pl.BlockSpec((HE, HE), lambda i: (0, 0)),
        ],
        out_specs=pl.BlockSpec((R, HE), lambda i: (i, 0)),
        scratch_shapes=[
            pltpu.VMEM((2, R, E), jnp.float32),
            pltpu.SemaphoreType.DMA((2, _NQ)),
        ],
        compiler_params=pltpu.CompilerParams(dimension_semantics=("arbitrary",)),
        cost_estimate=pl.CostEstimate(
            flops=2 * L * E * HE + 2 * L * HE * HE,
            transcendentals=L * HE,
            bytes_accessed=(L * E + L * HE) * 4 + (E * HE + HE * HE) * 4,
        ),
    )(label_emb, w1_cat, w2_bd)
